# baseline (speedup 1.0000x reference)
import functools

import jax
import jax.numpy as jnp
from jax import lax
from jax.experimental import pallas as pl
from jax.experimental.pallas import tpu as pltpu
from jax.experimental.pallas import tpu_sc as plsc

B = 16384
UNIFIED = 64
DIM_MID = 32
DIM_TAIL = 16
NC = 2
NS = 16
NW = NC * NS
RPW = B // NW
GCH = 128
NCH = RPW // GCH


def _body(xr_hbm, gr_hbm, head_hbm, mid_hbm, tail_hbm, wtm_hbm, bm_hbm,
          wtt_hbm, bt_hbm, out_hbm,
          xidx, gv, bufh, bufm, buft, outb, wtmv, bmv, wttv, btv, sem):
    wid = lax.axis_index("s") * NC + lax.axis_index("c")

    pltpu.sync_copy(xr_hbm.at[wid], xidx)
    pltpu.sync_copy(gr_hbm.at[wid], gv.at[pl.ds(0, RPW)])
    pltpu.sync_copy(wtm_hbm, wtmv)
    pltpu.sync_copy(bm_hbm, bmv)
    pltpu.sync_copy(wtt_hbm, wttv)
    pltpu.sync_copy(bt_hbm, btv)

    cps = []
    for j in range(NCH):
        cps.append(pltpu.async_copy(
            head_hbm.at[xidx.at[j]], bufh.at[pl.ds(j * GCH, GCH)], sem))
        cps.append(pltpu.async_copy(
            mid_hbm.at[xidx.at[j]], bufm.at[pl.ds(j * GCH, GCH)], sem))
        cps.append(pltpu.async_copy(
            tail_hbm.at[xidx.at[j]], buft.at[pl.ds(j * GCH, GCH)], sem))
    for c in cps:
        c.wait()

    def row(r, carry):
        g = gv[pl.ds(r, 16)][0]

        @pl.when(g == 0)
        def _():
            for j in range(4):
                outb[r, pl.ds(j * 16, 16)] = bufh[r, pl.ds(j * 16, 16)]

        @pl.when(g == 1)
        def _():
            ev = [bufm[r, pl.ds(h * 16, 16)] for h in range(DIM_MID // 16)]
            acc = [bmv[pl.ds(j * 16, 16)] for j in range(4)]
            for k in range(DIM_MID):
                e = ev[k // 16][k % 16]
                for j in range(4):
                    acc[j] = acc[j] + e * wtmv[k, pl.ds(j * 16, 16)]
            for j in range(4):
                outb[r, pl.ds(j * 16, 16)] = acc[j]

        @pl.when(g == 2)
        def _():
            ev = [buft[r, pl.ds(h * 16, 16)] for h in range(DIM_TAIL // 16)]
            acc = [btv[pl.ds(j * 16, 16)] for j in range(4)]
            for k in range(DIM_TAIL):
                e = ev[k // 16][k % 16]
                for j in range(4):
                    acc[j] = acc[j] + e * wttv[k, pl.ds(j * 16, 16)]
            for j in range(4):
                outb[r, pl.ds(j * 16, 16)] = acc[j]

        return carry

    lax.fori_loop(0, RPW, row, 0)
    pltpu.sync_copy(outb, out_hbm.at[wid])


@jax.jit
def _run(xr, gr, head_table, mid_table, tail_table, wtm, b_mid, wtt, b_tail):
    mesh = plsc.VectorSubcoreMesh(core_axis_name="c", subcore_axis_name="s")
    f = functools.partial(
        pl.kernel,
        mesh=mesh,
        compiler_params=pltpu.CompilerParams(use_tc_tiling_on_sc=False),
        out_type=jax.ShapeDtypeStruct((NW, RPW, UNIFIED), jnp.float32),
        scratch_types=[
            pltpu.VMEM((NCH, GCH), jnp.int32),
            pltpu.VMEM((RPW + 16,), jnp.int32),
            pltpu.VMEM((RPW, UNIFIED), jnp.float32),
            pltpu.VMEM((RPW, DIM_MID), jnp.float32),
            pltpu.VMEM((RPW, DIM_TAIL), jnp.float32),
            pltpu.VMEM((RPW, UNIFIED), jnp.float32),
            pltpu.VMEM((DIM_MID, UNIFIED), jnp.float32),
            pltpu.VMEM((UNIFIED,), jnp.float32),
            pltpu.VMEM((DIM_TAIL, UNIFIED), jnp.float32),
            pltpu.VMEM((UNIFIED,), jnp.float32),
            pltpu.SemaphoreType.DMA,
        ],
    )(_body)
    return f(xr, gr, head_table, mid_table, tail_table, wtm, b_mid, wtt,
             b_tail)


def kernel(x, frequency_groups, head_table, mid_table, tail_table, W_mid,
           b_mid, W_tail, b_tail):
    xr = x.reshape(NW, NCH, GCH)
    gr = frequency_groups.reshape(NW, RPW)
    wtm = W_mid.T
    wtt = W_tail.T
    out = _run(xr, gr, head_table, mid_table, tail_table, wtm, b_mid, wtt,
               b_tail)
    return out.reshape(B, UNIFIED)
